# baseline (speedup 1.0000x reference)
import functools

import jax
import jax.numpy as jnp
from jax import lax
from jax.experimental import pallas as pl
from jax.experimental.pallas import tpu as pltpu
from jax.experimental.pallas import tpu_sc as plsc

N, E, D = 10000, 320000, 128

_NC, _NS = 2, 16
_NW = _NC * _NS
_CH = 88
_G = 4
_NG = 29
_KJ = _G * _NG
_EPAD = _NW * _KJ * _CH
_NPAD = 10112
_RPT = _NPAD // _NS


def _sc_segment_sum(sd_hbm, x_hbm, zeros_hbm, out_hbm,
                    ib, rows, acc, gsem0, gsem1, gsem2, gsem3, isem):
    cid = lax.axis_index("c")
    sid = lax.axis_index("s")
    w = cid * _NS + sid
    gsems = (gsem0, gsem1, gsem2, gsem3)
    pltpu.sync_copy(zeros_hbm, acc.at[pl.ds(sid * _RPT, _RPT)])
    plsc.subcore_barrier()

    pltpu.sync_copy(sd_hbm.at[w].at[0], ib.at[0])
    for i in range(_G):
        pltpu.async_copy(x_hbm.at[ib.at[0, i, 0]], rows.at[i], gsems[i])
    pltpu.async_copy(sd_hbm.at[w].at[1], ib.at[1], isem)

    def body(g, carry):
        cur = lax.rem(g, 2)
        nxt = lax.rem(g + 1, 2)

        @pl.when(g + 1 < _NG)
        def _():
            pltpu.make_async_copy(sd_hbm.at[w].at[g + 1], ib.at[nxt],
                                  isem).wait()

        for i in range(_G):
            pltpu.make_async_copy(x_hbm.at[ib.at[cur, i, 0]],
                                  rows.at[i], gsems[i]).wait()
            pltpu.sync_copy(rows.at[i], acc.at[ib.at[cur, i, 1]], add=True)

            @pl.when(g + 1 < _NG)
            def _():
                pltpu.async_copy(x_hbm.at[ib.at[nxt, i, 0]], rows.at[i],
                                 gsems[i])

        @pl.when(g + 2 < _NG)
        def _():
            pltpu.async_copy(sd_hbm.at[w].at[g + 2], ib.at[cur], isem)

        return carry

    lax.fori_loop(0, _NG, body, 0)
    plsc.subcore_barrier()
    pltpu.sync_copy(acc.at[pl.ds(sid * _RPT, _RPT)],
                    out_hbm.at[cid].at[pl.ds(sid * _RPT, _RPT)])


_sc_seg = functools.partial(
    pl.kernel,
    out_type=jax.ShapeDtypeStruct((_NC, _NPAD, D), jnp.float32),
    mesh=plsc.VectorSubcoreMesh(core_axis_name="c", subcore_axis_name="s"),
    scratch_types=[
        pltpu.VMEM((2, _G, 2, _CH), jnp.int32),
        pltpu.VMEM((_G, _CH, D), jnp.float32),
        pltpu.VMEM_SHARED((_NPAD, D), jnp.float32),
        pltpu.SemaphoreType.DMA,
        pltpu.SemaphoreType.DMA,
        pltpu.SemaphoreType.DMA,
        pltpu.SemaphoreType.DMA,
        pltpu.SemaphoreType.DMA,
    ],
)(_sc_segment_sum)


def _tc_dense(x_ref, p_ref, w1_ref, b1_ref, w2_ref, b2_ref, eps_ref,
              gamma_ref, beta_ref, o_ref):
    x = x_ref[...]
    h = (1.0 + eps_ref[0, 0]) * x + p_ref[0, :N, :] + p_ref[1, :N, :]
    h = lax.dot_general(h, w1_ref[...], (((1,), (1,)), ((), ())),
                        preferred_element_type=jnp.float32) + b1_ref[...]
    h = jnp.maximum(h, 0.0)
    h = lax.dot_general(h, w2_ref[...], (((1,), (1,)), ((), ())),
                        preferred_element_type=jnp.float32) + b2_ref[...]
    mean = jnp.mean(h, axis=0, keepdims=True)
    var = jnp.mean((h - mean) ** 2, axis=0, keepdims=True)
    o_ref[...] = (h - mean) * lax.rsqrt(var + 1e-5) * gamma_ref[...] \
        + beta_ref[...] + x


def kernel(x, edge_index, W1, b1, W2, b2, eps, gamma, beta):
    src = edge_index[0]
    dst = edge_index[1]
    pad = _EPAD - E
    x_pad = jnp.concatenate([x, jnp.zeros((8, D), x.dtype)], axis=0)
    src_p = jnp.concatenate(
        [src, jnp.full((pad,), N, jnp.int32)]
    ).reshape(_KJ, _NW, _CH).transpose(1, 0, 2)
    pad_dst = N + jnp.arange(pad, dtype=jnp.int32) % (_NPAD - N)
    dst_p = jnp.concatenate(
        [dst, pad_dst]).reshape(_KJ, _NW, _CH).transpose(1, 0, 2)
    sd = jnp.stack([src_p, dst_p], axis=2).reshape(_NW, _NG, _G, 2, _CH)
    zeros = jnp.zeros((_RPT, D), jnp.float32)

    partials = _sc_seg(sd, x_pad, zeros)

    out = pl.pallas_call(
        _tc_dense,
        out_shape=jax.ShapeDtypeStruct((N, D), jnp.float32),
    )(x, partials, W1, b1.reshape(1, D), W2, b2.reshape(1, D),
      eps.reshape(1, 1), gamma.reshape(1, D), beta.reshape(1, D))
    return out


# speedup vs baseline: 1.6123x; 1.6123x over previous
import functools

import jax
import jax.numpy as jnp
from jax import lax
from jax.experimental import pallas as pl
from jax.experimental.pallas import tpu as pltpu
from jax.experimental.pallas import tpu_sc as plsc

N, E, D = 10000, 320000, 128

_NC, _NS = 2, 16
_NW = _NC * _NS
_CH = 120
_G = 3
_NG = 28
_KJ = _G * _NG
_EPAD = _NW * _KJ * _CH
_NPAD = 10112
_RPT = _NPAD // _NS


def _sc_segment_sum(sd_hbm, x_hbm, zeros_hbm, out_hbm,
                    ib, rows, acc, gsem0, gsem1, gsem2, isem):
    cid = lax.axis_index("c")
    sid = lax.axis_index("s")
    w = cid * _NS + sid
    gsems = (gsem0, gsem1, gsem2)
    pltpu.sync_copy(zeros_hbm, acc.at[pl.ds(sid * _RPT, _RPT)])
    plsc.subcore_barrier()

    pltpu.sync_copy(sd_hbm.at[w].at[0], ib.at[0])
    for i in range(_G):
        pltpu.async_copy(x_hbm.at[ib.at[0, i, 0]], rows.at[i], gsems[i])
    pltpu.async_copy(sd_hbm.at[w].at[1], ib.at[1], isem)

    def body(g, carry):
        cur = lax.rem(g, 2)
        nxt = lax.rem(g + 1, 2)

        @pl.when(g + 1 < _NG)
        def _():
            pltpu.make_async_copy(sd_hbm.at[w].at[g + 1], ib.at[nxt],
                                  isem).wait()

        for i in range(_G):
            pltpu.make_async_copy(x_hbm.at[ib.at[cur, i, 0]],
                                  rows.at[i], gsems[i]).wait()
            pltpu.sync_copy(rows.at[i], acc.at[ib.at[cur, i, 1]], add=True)

            @pl.when(g + 1 < _NG)
            def _():
                pltpu.async_copy(x_hbm.at[ib.at[nxt, i, 0]], rows.at[i],
                                 gsems[i])

        @pl.when(g + 2 < _NG)
        def _():
            pltpu.async_copy(sd_hbm.at[w].at[g + 2], ib.at[cur], isem)

        return carry

    lax.fori_loop(0, _NG, body, 0)
    plsc.subcore_barrier()
    pltpu.sync_copy(acc.at[pl.ds(sid * _RPT, _RPT)],
                    out_hbm.at[cid].at[pl.ds(sid * _RPT, _RPT)])


_sc_seg = functools.partial(
    pl.kernel,
    out_type=jax.ShapeDtypeStruct((_NC, _NPAD, D), jnp.float32),
    mesh=plsc.VectorSubcoreMesh(core_axis_name="c", subcore_axis_name="s"),
    scratch_types=[
        pltpu.VMEM((2, _G, 2, _CH), jnp.int32),
        pltpu.VMEM((_G, _CH, D), jnp.float32),
        pltpu.VMEM_SHARED((_NPAD, D), jnp.float32),
        pltpu.SemaphoreType.DMA,
        pltpu.SemaphoreType.DMA,
        pltpu.SemaphoreType.DMA,
        pltpu.SemaphoreType.DMA,
    ],
)(_sc_segment_sum)


def _tc_dense(x_ref, p_ref, w1_ref, b1_ref, w2_ref, b2_ref, eps_ref,
              gamma_ref, beta_ref, o_ref):
    x = x_ref[...]
    h = (1.0 + eps_ref[0, 0]) * x + p_ref[0, :N, :] + p_ref[1, :N, :]
    h = lax.dot_general(h, w1_ref[...], (((1,), (1,)), ((), ())),
                        preferred_element_type=jnp.float32) + b1_ref[...]
    h = jnp.maximum(h, 0.0)
    h = lax.dot_general(h, w2_ref[...], (((1,), (1,)), ((), ())),
                        preferred_element_type=jnp.float32) + b2_ref[...]
    mean = jnp.mean(h, axis=0, keepdims=True)
    var = jnp.mean((h - mean) ** 2, axis=0, keepdims=True)
    o_ref[...] = (h - mean) * lax.rsqrt(var + 1e-5) * gamma_ref[...] \
        + beta_ref[...] + x


def kernel(x, edge_index, W1, b1, W2, b2, eps, gamma, beta):
    src = edge_index[0]
    dst = edge_index[1]
    pad = _EPAD - E
    x_pad = jnp.concatenate([x, jnp.zeros((8, D), x.dtype)], axis=0)
    src_p = jnp.concatenate(
        [src, jnp.full((pad,), N, jnp.int32)]
    ).reshape(_KJ, _NW, _CH).transpose(1, 0, 2)
    pad_dst = N + jnp.arange(pad, dtype=jnp.int32) % (_NPAD - N)
    dst_p = jnp.concatenate(
        [dst, pad_dst]).reshape(_KJ, _NW, _CH).transpose(1, 0, 2)
    sd = jnp.stack([src_p, dst_p], axis=2).reshape(_NW, _NG, _G, 2, _CH)
    zeros = jnp.zeros((_RPT, D), jnp.float32)

    partials = _sc_seg(sd, x_pad, zeros)

    out = pl.pallas_call(
        _tc_dense,
        out_shape=jax.ShapeDtypeStruct((N, D), jnp.float32),
    )(x, partials, W1, b1.reshape(1, D), W2, b2.reshape(1, D),
      eps.reshape(1, 1), gamma.reshape(1, D), beta.reshape(1, D))
    return out
